# Initial kernel scaffold; baseline (speedup 1.0000x reference)
import jax
import jax.numpy as jnp
from jax.experimental import pallas as pl


def kernel(x, edge_index, W, b):
    raise NotImplementedError("write your pallas kernel here")


# speedup vs baseline: 1.3214x; 1.3214x over previous
import functools

import jax
import jax.numpy as jnp
from jax import lax
from jax.experimental import pallas as pl
from jax.experimental.pallas import tpu as pltpu
from jax.experimental.pallas import tpu_sc as plsc

N = 10000
E = 320000
D = 128
HOPS = 3
SIGMA = 0.1

NC = 2
NS = 16
L = 16
DH = D // NC

E_PER_T = E // NS
CHUNK = 400
N_CHUNKS = E_PER_T // CHUNK
GROUPS = CHUNK // L
ROWS_PER_TILE = 624
ROWS_LAST_EXTRA = N - NS * ROWS_PER_TILE

SELU_ALPHA = 1.6732632423543772
SELU_SCALE = 1.0507009873554805


def _tc_stage_body(p_ref, nz_ref, w_ref, bk_ref, h_ref, hs_ref, s1_ref,
                   s2_ref):
    agg = jnp.concatenate([p_ref[0], p_ref[1]], axis=1) + nz_ref[...]
    nrm = jnp.sqrt(jnp.sum(agg * agg, axis=1, keepdims=True))
    h = agg / jnp.maximum(nrm, 1e-12)
    h_ref[...] = h
    hs_ref[0] = h[:, :DH]
    hs_ref[1] = h[:, DH:]
    s1_ref[...] = jnp.sum(h * w_ref[0:1, :], axis=1)
    s2_ref[...] = jnp.sum(h * w_ref[1:2, :], axis=1) + bk_ref[0, 0]


def _tc_stage(p, nz, wk, bk):
    return pl.pallas_call(
        _tc_stage_body,
        out_shape=(
            jax.ShapeDtypeStruct((N, D), jnp.float32),
            jax.ShapeDtypeStruct((NC, N, DH), jnp.float32),
            jax.ShapeDtypeStruct((N,), jnp.float32),
            jax.ShapeDtypeStruct((N,), jnp.float32),
        ),
        in_specs=[
            pl.BlockSpec(memory_space=pltpu.VMEM),
            pl.BlockSpec(memory_space=pltpu.VMEM),
            pl.BlockSpec(memory_space=pltpu.VMEM),
            pl.BlockSpec(memory_space=pltpu.SMEM),
        ],
        out_specs=(
            pl.BlockSpec(memory_space=pltpu.VMEM),
            pl.BlockSpec(memory_space=pltpu.VMEM),
            pl.BlockSpec(memory_space=pltpu.VMEM),
            pl.BlockSpec(memory_space=pltpu.VMEM),
        ),
    )(p, nz, wk, bk)


def _sc_hop_body(hs_hbm, s1_hbm, s2_hbm, src_hbm, dst_hbm, part_hbm,
                 s1_v, s2_v, src_v, dst_v, rows_v, aggr_sh, sem):
    cid = lax.axis_index("c")
    sid = lax.axis_index("s")

    zero16 = jnp.zeros((L,), jnp.float32)

    def zbody(j, _):
        for cc in range(DH // L):
            rows_v[j, pl.ds(cc * L, L)] = zero16
        return 0

    lax.fori_loop(0, CHUNK, zbody, 0)
    row0 = sid * ROWS_PER_TILE
    pltpu.sync_copy(rows_v.at[pl.ds(0, CHUNK)], aggr_sh.at[pl.ds(row0, CHUNK)])
    pltpu.sync_copy(rows_v.at[pl.ds(0, ROWS_PER_TILE - CHUNK)],
                    aggr_sh.at[pl.ds(row0 + CHUNK, ROWS_PER_TILE - CHUNK)])

    @pl.when(sid == NS - 1)
    def _zero_tail():
        pltpu.sync_copy(rows_v.at[pl.ds(0, ROWS_LAST_EXTRA)],
                        aggr_sh.at[pl.ds(NS * ROWS_PER_TILE, ROWS_LAST_EXTRA)])

    pltpu.sync_copy(s1_hbm, s1_v)
    pltpu.sync_copy(s2_hbm, s2_v)
    plsc.subcore_barrier()

    iota = lax.iota(jnp.int32, L)

    def chunk_body(i, _):
        off = sid * E_PER_T + i * CHUNK
        pltpu.sync_copy(src_hbm.at[pl.ds(off, CHUNK)], src_v)
        pltpu.sync_copy(dst_hbm.at[pl.ds(off, CHUNK)], dst_v)
        pltpu.async_copy(hs_hbm.at[cid].at[src_v], rows_v, sem).wait()

        def group_body(g, _):
            base = g * L
            srcg = src_v[pl.ds(base, L)]
            dstg = dst_v[pl.ds(base, L)]
            a = plsc.load_gather(s1_v, [srcg]) + plsc.load_gather(s2_v, [dstg])
            selu = SELU_SCALE * jnp.where(
                a > 0.0, a, SELU_ALPHA * (jnp.exp(a) - 1.0))
            alpha = 1.0 / (1.0 + jnp.exp(-selu))
            jrow = base + iota
            for col in range(DH):
                cidx = jnp.full((L,), col, jnp.int32)
                v = plsc.load_gather(rows_v, [jrow, cidx])
                plsc.store_scatter(rows_v, [jrow, cidx], v * alpha)
            return 0

        lax.fori_loop(0, GROUPS, group_body, 0)
        pltpu.sync_copy(rows_v, aggr_sh.at[dst_v], add=True)
        return 0

    lax.fori_loop(0, N_CHUNKS, chunk_body, 0)
    plsc.subcore_barrier()

    pltpu.sync_copy(aggr_sh.at[pl.ds(row0, ROWS_PER_TILE)],
                    part_hbm.at[cid, pl.ds(row0, ROWS_PER_TILE)])

    @pl.when(sid == NS - 1)
    def _write_tail():
        pltpu.sync_copy(aggr_sh.at[pl.ds(NS * ROWS_PER_TILE, ROWS_LAST_EXTRA)],
                        part_hbm.at[cid, pl.ds(NS * ROWS_PER_TILE,
                                               ROWS_LAST_EXTRA)])


_sc_hop = functools.partial(
    pl.kernel,
    out_type=jax.ShapeDtypeStruct((NC, N, DH), jnp.float32),
    mesh=plsc.VectorSubcoreMesh(core_axis_name="c", subcore_axis_name="s"),
    scratch_types=[
        pltpu.VMEM((N,), jnp.float32),
        pltpu.VMEM((N,), jnp.float32),
        pltpu.VMEM((CHUNK,), jnp.int32),
        pltpu.VMEM((CHUNK,), jnp.int32),
        pltpu.VMEM((CHUNK, DH), jnp.float32),
        pltpu.VMEM_SHARED((N, DH), jnp.float32),
        pltpu.SemaphoreType.DMA,
    ],
    compiler_params=pltpu.CompilerParams(needs_layout_passes=False,
                                         use_tc_tiling_on_sc=False),
)(_sc_hop_body)


@jax.jit
def kernel(x, edge_index, W, b):
    src = edge_index[0]
    dst = edge_index[1]
    zeros_nd = jnp.zeros((N, D), jnp.float32)
    xsplit = jnp.stack([x[:, :DH], x[:, DH:]])

    noises = [
        SIGMA * jax.random.normal(
            jax.random.fold_in(jax.random.key(1), k), (N, D), dtype=jnp.float32)
        for k in range(HOPS)
    ]

    outs = []
    p, nz = xsplit, zeros_nd
    for k in range(HOPS + 1):
        wk = W[min(k, HOPS - 1)].reshape(2, D)
        bk = b[min(k, HOPS - 1)].reshape(1, 1)
        h, hs, s1, s2 = _tc_stage(p, nz, wk, bk)
        outs.append(h)
        if k == HOPS:
            break
        p = _sc_hop(hs, s1, s2, src, dst)
        nz = noises[k]

    return jnp.stack(outs)


# speedup vs baseline: 3.5419x; 2.6805x over previous
import functools

import jax
import jax.numpy as jnp
from jax import lax
from jax.experimental import pallas as pl
from jax.experimental.pallas import tpu as pltpu
from jax.experimental.pallas import tpu_sc as plsc

N = 10000
E = 320000
D = 128
HOPS = 3
SIGMA = 0.1

NC = 2
NS = 16
L = 16
DH = D // NC

E_PER_T = E // NS
CHUNK = 400
N_CHUNKS = E_PER_T // CHUNK
GROUPS = CHUNK // L
ROWS_PER_TILE = 624
ROWS_LAST_EXTRA = N - NS * ROWS_PER_TILE

SELU_ALPHA = 1.6732632423543772
SELU_SCALE = 1.0507009873554805


def _tc_stage_body(p_ref, nz_ref, w_ref, bk_ref, h_ref, hs_ref, s1_ref,
                   s2_ref):
    agg = jnp.concatenate([p_ref[0], p_ref[1]], axis=1) + nz_ref[...]
    nrm = jnp.sqrt(jnp.sum(agg * agg, axis=1, keepdims=True))
    h = agg / jnp.maximum(nrm, 1e-12)
    h_ref[...] = h
    hs_ref[0] = h[:, :DH]
    hs_ref[1] = h[:, DH:]
    s1_ref[...] = jnp.sum(h * w_ref[0:1, :], axis=1)
    s2_ref[...] = jnp.sum(h * w_ref[1:2, :], axis=1) + bk_ref[0, 0]


def _tc_stage(p, nz, wk, bk):
    return pl.pallas_call(
        _tc_stage_body,
        out_shape=(
            jax.ShapeDtypeStruct((N, D), jnp.float32),
            jax.ShapeDtypeStruct((NC, N, DH), jnp.float32),
            jax.ShapeDtypeStruct((N,), jnp.float32),
            jax.ShapeDtypeStruct((N,), jnp.float32),
        ),
        in_specs=[
            pl.BlockSpec(memory_space=pltpu.VMEM),
            pl.BlockSpec(memory_space=pltpu.VMEM),
            pl.BlockSpec(memory_space=pltpu.VMEM),
            pl.BlockSpec(memory_space=pltpu.SMEM),
        ],
        out_specs=(
            pl.BlockSpec(memory_space=pltpu.VMEM),
            pl.BlockSpec(memory_space=pltpu.VMEM),
            pl.BlockSpec(memory_space=pltpu.VMEM),
            pl.BlockSpec(memory_space=pltpu.VMEM),
        ),
    )(p, nz, wk, bk)


def _sc_hop_body(hs_hbm, s1_hbm, s2_hbm, src_hbm, dst_hbm, part_hbm,
                 s1_v, s2_v, src_v, dst_v, rows_v, aggr_sh, sem):
    cid = lax.axis_index("c")
    sid = lax.axis_index("s")

    zero16 = jnp.zeros((L,), jnp.float32)

    def zbody(j, _):
        for cc in range(DH // L):
            rows_v[j, pl.ds(cc * L, L)] = zero16
        return 0

    lax.fori_loop(0, CHUNK, zbody, 0)
    row0 = sid * ROWS_PER_TILE
    pltpu.sync_copy(rows_v.at[pl.ds(0, CHUNK)], aggr_sh.at[pl.ds(row0, CHUNK)])
    pltpu.sync_copy(rows_v.at[pl.ds(0, ROWS_PER_TILE - CHUNK)],
                    aggr_sh.at[pl.ds(row0 + CHUNK, ROWS_PER_TILE - CHUNK)])

    @pl.when(sid == NS - 1)
    def _zero_tail():
        pltpu.sync_copy(rows_v.at[pl.ds(0, ROWS_LAST_EXTRA)],
                        aggr_sh.at[pl.ds(NS * ROWS_PER_TILE, ROWS_LAST_EXTRA)])

    pltpu.sync_copy(s1_hbm, s1_v)
    pltpu.sync_copy(s2_hbm, s2_v)
    plsc.subcore_barrier()

    iota = lax.iota(jnp.int32, L)

    def chunk_body(i, _):
        off = sid * E_PER_T + i * CHUNK
        pltpu.sync_copy(src_hbm.at[pl.ds(off, CHUNK)], src_v)
        pltpu.sync_copy(dst_hbm.at[pl.ds(off, CHUNK)], dst_v)
        pltpu.async_copy(hs_hbm.at[cid].at[src_v], rows_v, sem).wait()

        def group_body(g, _):
            base = g * L
            srcg = src_v[pl.ds(base, L)]
            dstg = dst_v[pl.ds(base, L)]
            a = plsc.load_gather(s1_v, [srcg]) + plsc.load_gather(s2_v, [dstg])
            selu = SELU_SCALE * jnp.where(
                a > 0.0, a, SELU_ALPHA * (jnp.exp(a) - 1.0))
            alpha = 1.0 / (1.0 + jnp.exp(-selu))
            jrow = base + iota
            for col in range(DH):
                cidx = jnp.bitwise_and(col + iota, DH - 1)
                v = plsc.load_gather(rows_v, [jrow, cidx])
                plsc.store_scatter(rows_v, [jrow, cidx], v * alpha)
            return 0

        lax.fori_loop(0, GROUPS, group_body, 0)
        pltpu.sync_copy(rows_v, aggr_sh.at[dst_v], add=True)
        return 0

    lax.fori_loop(0, N_CHUNKS, chunk_body, 0)
    plsc.subcore_barrier()

    pltpu.sync_copy(aggr_sh.at[pl.ds(row0, ROWS_PER_TILE)],
                    part_hbm.at[cid, pl.ds(row0, ROWS_PER_TILE)])

    @pl.when(sid == NS - 1)
    def _write_tail():
        pltpu.sync_copy(aggr_sh.at[pl.ds(NS * ROWS_PER_TILE, ROWS_LAST_EXTRA)],
                        part_hbm.at[cid, pl.ds(NS * ROWS_PER_TILE,
                                               ROWS_LAST_EXTRA)])


_sc_hop = functools.partial(
    pl.kernel,
    out_type=jax.ShapeDtypeStruct((NC, N, DH), jnp.float32),
    mesh=plsc.VectorSubcoreMesh(core_axis_name="c", subcore_axis_name="s"),
    scratch_types=[
        pltpu.VMEM((N,), jnp.float32),
        pltpu.VMEM((N,), jnp.float32),
        pltpu.VMEM((CHUNK,), jnp.int32),
        pltpu.VMEM((CHUNK,), jnp.int32),
        pltpu.VMEM((CHUNK, DH), jnp.float32),
        pltpu.VMEM_SHARED((N, DH), jnp.float32),
        pltpu.SemaphoreType.DMA,
    ],
    compiler_params=pltpu.CompilerParams(needs_layout_passes=False,
                                         use_tc_tiling_on_sc=False),
)(_sc_hop_body)


@jax.jit
def kernel(x, edge_index, W, b):
    src = edge_index[0]
    dst = edge_index[1]
    zeros_nd = jnp.zeros((N, D), jnp.float32)
    xsplit = jnp.stack([x[:, :DH], x[:, DH:]])

    noises = [
        SIGMA * jax.random.normal(
            jax.random.fold_in(jax.random.key(1), k), (N, D), dtype=jnp.float32)
        for k in range(HOPS)
    ]

    outs = []
    p, nz = xsplit, zeros_nd
    for k in range(HOPS + 1):
        wk = W[min(k, HOPS - 1)].reshape(2, D)
        bk = b[min(k, HOPS - 1)].reshape(1, 1)
        h, hs, s1, s2 = _tc_stage(p, nz, wk, bk)
        outs.append(h)
        if k == HOPS:
            break
        p = _sc_hop(hs, s1, s2, src, dst)
        nz = noises[k]

    return jnp.stack(outs)


# speedup vs baseline: 4.3303x; 1.2226x over previous
import functools

import jax
import jax.numpy as jnp
from jax import lax
from jax.experimental import pallas as pl
from jax.experimental.pallas import tpu as pltpu
from jax.experimental.pallas import tpu_sc as plsc

N = 10000
E = 320000
D = 128
HOPS = 3
SIGMA = 0.1

NC = 2
NS = 16
L = 16
DH = D // NC

E_PER_T = E // NS
CHUNK = 400
N_CHUNKS = E_PER_T // CHUNK
N_PAIRS = N_CHUNKS // 2
GROUPS = CHUNK // L
ROWS_PER_TILE = 624
ROWS_LAST_EXTRA = N - NS * ROWS_PER_TILE

SELU_ALPHA = 1.6732632423543772
SELU_SCALE = 1.0507009873554805


def _tc_stage_body(p_ref, nz_ref, w_ref, bk_ref, h_ref, hs_ref, s1_ref,
                   s2_ref):
    agg = jnp.concatenate([p_ref[0], p_ref[1]], axis=1) + nz_ref[...]
    nrm = jnp.sqrt(jnp.sum(agg * agg, axis=1, keepdims=True))
    h = agg / jnp.maximum(nrm, 1e-12)
    h_ref[...] = h
    hs_ref[0] = h[:, :DH]
    hs_ref[1] = h[:, DH:]
    s1_ref[...] = jnp.sum(h * w_ref[0:1, :], axis=1)
    s2_ref[...] = jnp.sum(h * w_ref[1:2, :], axis=1) + bk_ref[0, 0]


def _tc_stage(p, nz, wk, bk):
    return pl.pallas_call(
        _tc_stage_body,
        out_shape=(
            jax.ShapeDtypeStruct((N, D), jnp.float32),
            jax.ShapeDtypeStruct((NC, N, DH), jnp.float32),
            jax.ShapeDtypeStruct((N,), jnp.float32),
            jax.ShapeDtypeStruct((N,), jnp.float32),
        ),
        in_specs=[
            pl.BlockSpec(memory_space=pltpu.VMEM),
            pl.BlockSpec(memory_space=pltpu.VMEM),
            pl.BlockSpec(memory_space=pltpu.VMEM),
            pl.BlockSpec(memory_space=pltpu.SMEM),
        ],
        out_specs=(
            pl.BlockSpec(memory_space=pltpu.VMEM),
            pl.BlockSpec(memory_space=pltpu.VMEM),
            pl.BlockSpec(memory_space=pltpu.VMEM),
            pl.BlockSpec(memory_space=pltpu.VMEM),
        ),
    )(p, nz, wk, bk)


def _sc_hop_body(hs_hbm, s1_hbm, s2_hbm, src_hbm, dst_hbm, part_hbm,
                 s1_v, s2_v, src0_v, dst0_v, src1_v, dst1_v,
                 rows0_v, rows1_v, aggr_sh,
                 gsem0, gsem1, ssem0, ssem1):
    cid = lax.axis_index("c")
    sid = lax.axis_index("s")
    bufs = ((src0_v, dst0_v, rows0_v, gsem0, ssem0),
            (src1_v, dst1_v, rows1_v, gsem1, ssem1))

    zero16 = jnp.zeros((L,), jnp.float32)

    def zbody(j, _):
        for cc in range(DH // L):
            rows0_v[j, pl.ds(cc * L, L)] = zero16
        return 0

    lax.fori_loop(0, CHUNK, zbody, 0)
    row0 = sid * ROWS_PER_TILE
    pltpu.sync_copy(rows0_v.at[pl.ds(0, CHUNK)], aggr_sh.at[pl.ds(row0, CHUNK)])
    pltpu.sync_copy(rows0_v.at[pl.ds(0, ROWS_PER_TILE - CHUNK)],
                    aggr_sh.at[pl.ds(row0 + CHUNK, ROWS_PER_TILE - CHUNK)])

    @pl.when(sid == NS - 1)
    def _zero_tail():
        pltpu.sync_copy(rows0_v.at[pl.ds(0, ROWS_LAST_EXTRA)],
                        aggr_sh.at[pl.ds(NS * ROWS_PER_TILE, ROWS_LAST_EXTRA)])

    pltpu.sync_copy(s1_hbm, s1_v)
    pltpu.sync_copy(s2_hbm, s2_v)
    plsc.subcore_barrier()

    iota = lax.iota(jnp.int32, L)
    ebase = sid * E_PER_T

    def fetch(p, chunk_idx):
        src_v, dst_v, rows_v, gsem, _ = bufs[p]
        off = ebase + chunk_idx * CHUNK
        pltpu.sync_copy(src_hbm.at[pl.ds(off, CHUNK)], src_v)
        pltpu.sync_copy(dst_hbm.at[pl.ds(off, CHUNK)], dst_v)
        pltpu.async_copy(hs_hbm.at[cid].at[src_v], rows_v, gsem)

    def drain_scatter(p):
        _, _, rows_v, _, ssem = bufs[p]
        pltpu.make_async_copy(rows_v, aggr_sh.at[pl.ds(0, CHUNK)], ssem).wait()

    def process(p):
        src_v, dst_v, rows_v, gsem, ssem = bufs[p]
        pltpu.make_async_copy(
            hs_hbm.at[cid].at[src_v], rows_v, gsem).wait()

        def group_body(g, _):
            base = g * L
            srcg = src_v[pl.ds(base, L)]
            dstg = dst_v[pl.ds(base, L)]
            a = plsc.load_gather(s1_v, [srcg]) + plsc.load_gather(s2_v, [dstg])
            selu = SELU_SCALE * jnp.where(
                a > 0.0, a, SELU_ALPHA * (jnp.exp(a) - 1.0))
            alpha = 1.0 / (1.0 + jnp.exp(-selu))
            jrow = base + iota
            for col in range(DH):
                cidx = jnp.bitwise_and(col + iota, DH - 1)
                v = plsc.load_gather(rows_v, [jrow, cidx])
                plsc.store_scatter(rows_v, [jrow, cidx], v * alpha)
            return 0

        lax.fori_loop(0, GROUPS, group_body, 0)
        pltpu.async_copy(rows_v, aggr_sh.at[dst_v], ssem, add=True)

    fetch(0, 0)
    fetch(1, 1)

    def pair_body(i2, _):
        process(0)
        process(1)

        @pl.when(i2 < N_PAIRS - 1)
        def _prefetch():
            drain_scatter(0)
            fetch(0, 2 * i2 + 2)
            drain_scatter(1)
            fetch(1, 2 * i2 + 3)

        return 0

    lax.fori_loop(0, N_PAIRS, pair_body, 0)
    drain_scatter(0)
    drain_scatter(1)
    plsc.subcore_barrier()

    pltpu.sync_copy(aggr_sh.at[pl.ds(row0, ROWS_PER_TILE)],
                    part_hbm.at[cid, pl.ds(row0, ROWS_PER_TILE)])

    @pl.when(sid == NS - 1)
    def _write_tail():
        pltpu.sync_copy(aggr_sh.at[pl.ds(NS * ROWS_PER_TILE, ROWS_LAST_EXTRA)],
                        part_hbm.at[cid, pl.ds(NS * ROWS_PER_TILE,
                                               ROWS_LAST_EXTRA)])


_sc_hop = functools.partial(
    pl.kernel,
    out_type=jax.ShapeDtypeStruct((NC, N, DH), jnp.float32),
    mesh=plsc.VectorSubcoreMesh(core_axis_name="c", subcore_axis_name="s"),
    scratch_types=[
        pltpu.VMEM((N,), jnp.float32),
        pltpu.VMEM((N,), jnp.float32),
        pltpu.VMEM((CHUNK,), jnp.int32),
        pltpu.VMEM((CHUNK,), jnp.int32),
        pltpu.VMEM((CHUNK,), jnp.int32),
        pltpu.VMEM((CHUNK,), jnp.int32),
        pltpu.VMEM((CHUNK, DH), jnp.float32),
        pltpu.VMEM((CHUNK, DH), jnp.float32),
        pltpu.VMEM_SHARED((N, DH), jnp.float32),
        pltpu.SemaphoreType.DMA,
        pltpu.SemaphoreType.DMA,
        pltpu.SemaphoreType.DMA,
        pltpu.SemaphoreType.DMA,
    ],
    compiler_params=pltpu.CompilerParams(needs_layout_passes=False,
                                         use_tc_tiling_on_sc=False),
)(_sc_hop_body)


@jax.jit
def kernel(x, edge_index, W, b):
    src = edge_index[0]
    dst = edge_index[1]
    zeros_nd = jnp.zeros((N, D), jnp.float32)
    xsplit = jnp.stack([x[:, :DH], x[:, DH:]])

    noises = [
        SIGMA * jax.random.normal(
            jax.random.fold_in(jax.random.key(1), k), (N, D), dtype=jnp.float32)
        for k in range(HOPS)
    ]

    outs = []
    p, nz = xsplit, zeros_nd
    for k in range(HOPS + 1):
        wk = W[min(k, HOPS - 1)].reshape(2, D)
        bk = b[min(k, HOPS - 1)].reshape(1, 1)
        h, hs, s1, s2 = _tc_stage(p, nz, wk, bk)
        outs.append(h)
        if k == HOPS:
            break
        p = _sc_hop(hs, s1, s2, src, dst)
        nz = noises[k]

    return jnp.stack(outs)


# speedup vs baseline: 6.0646x; 1.4005x over previous
import functools

import jax
import jax.numpy as jnp
from jax import lax
from jax.experimental import pallas as pl
from jax.experimental.pallas import tpu as pltpu
from jax.experimental.pallas import tpu_sc as plsc

N = 10000
E = 320000
D = 128
HOPS = 3
SIGMA = 0.1

NC = 2
NS = 16
L = 16
DH = D // NC

E_PER_T = E // NS
CHUNK = 400
N_CHUNKS = E_PER_T // CHUNK
N_PAIRS = N_CHUNKS // 2
GROUPS = CHUNK // L
ROWS_PER_TILE = 624
ROWS_LAST_EXTRA = N - NS * ROWS_PER_TILE

SELU_ALPHA = 1.6732632423543772
SELU_SCALE = 1.0507009873554805


def _tc_stage_body(p_ref, nz_ref, w_ref, bk_ref, h_ref, hs_ref, s1_ref,
                   s2_ref):
    agg = jnp.concatenate([p_ref[0], p_ref[1]], axis=1) + nz_ref[...]
    nrm = jnp.sqrt(jnp.sum(agg * agg, axis=1, keepdims=True))
    h = agg / jnp.maximum(nrm, 1e-12)
    h_ref[...] = h
    hs_ref[0] = h[:, :DH]
    hs_ref[1] = h[:, DH:]
    s1_ref[...] = jnp.sum(h * w_ref[0:1, :], axis=1)
    s2_ref[...] = jnp.sum(h * w_ref[1:2, :], axis=1) + bk_ref[0, 0]


def _tc_stage(p, nz, wk, bk):
    return pl.pallas_call(
        _tc_stage_body,
        out_shape=(
            jax.ShapeDtypeStruct((N, D), jnp.float32),
            jax.ShapeDtypeStruct((NC, N, DH), jnp.float32),
            jax.ShapeDtypeStruct((N,), jnp.float32),
            jax.ShapeDtypeStruct((N,), jnp.float32),
        ),
        in_specs=[
            pl.BlockSpec(memory_space=pltpu.VMEM),
            pl.BlockSpec(memory_space=pltpu.VMEM),
            pl.BlockSpec(memory_space=pltpu.VMEM),
            pl.BlockSpec(memory_space=pltpu.SMEM),
        ],
        out_specs=(
            pl.BlockSpec(memory_space=pltpu.VMEM),
            pl.BlockSpec(memory_space=pltpu.VMEM),
            pl.BlockSpec(memory_space=pltpu.VMEM),
            pl.BlockSpec(memory_space=pltpu.VMEM),
        ),
    )(p, nz, wk, bk)


def _sc_hop_body(hs_hbm, s1_hbm, s2_hbm, src_hbm, dst_hbm, part_hbm,
                 s1_v, s2_v, src0_v, dst0_v, src1_v, dst1_v,
                 rows0_v, rows1_v, aggr_sh,
                 gsem0, gsem1, ssem0, ssem1):
    cid = lax.axis_index("c")
    sid = lax.axis_index("s")
    bufs = ((src0_v, dst0_v, rows0_v, gsem0, ssem0),
            (src1_v, dst1_v, rows1_v, gsem1, ssem1))

    zero16 = jnp.zeros((L,), jnp.float32)

    def zbody(j, _):
        for cc in range(DH // L):
            rows0_v[j, pl.ds(cc * L, L)] = zero16
        return 0

    lax.fori_loop(0, CHUNK, zbody, 0)
    row0 = sid * ROWS_PER_TILE
    pltpu.sync_copy(rows0_v.at[pl.ds(0, CHUNK)], aggr_sh.at[pl.ds(row0, CHUNK)])
    pltpu.sync_copy(rows0_v.at[pl.ds(0, ROWS_PER_TILE - CHUNK)],
                    aggr_sh.at[pl.ds(row0 + CHUNK, ROWS_PER_TILE - CHUNK)])

    @pl.when(sid == NS - 1)
    def _zero_tail():
        pltpu.sync_copy(rows0_v.at[pl.ds(0, ROWS_LAST_EXTRA)],
                        aggr_sh.at[pl.ds(NS * ROWS_PER_TILE, ROWS_LAST_EXTRA)])

    pltpu.sync_copy(s1_hbm, s1_v)
    pltpu.sync_copy(s2_hbm, s2_v)
    plsc.subcore_barrier()

    iota = lax.iota(jnp.int32, L)
    ebase = sid * E_PER_T

    def fetch(p, chunk_idx):
        src_v, dst_v, rows_v, gsem, _ = bufs[p]
        off = ebase + chunk_idx * CHUNK
        pltpu.sync_copy(src_hbm.at[pl.ds(off, CHUNK)], src_v)
        pltpu.sync_copy(dst_hbm.at[pl.ds(off, CHUNK)], dst_v)
        pltpu.async_copy(hs_hbm.at[cid].at[src_v], rows_v, gsem)

    def drain_scatter(p):
        _, _, rows_v, _, ssem = bufs[p]
        pltpu.make_async_copy(rows_v, aggr_sh.at[pl.ds(0, CHUNK)], ssem).wait()

    def process(p):
        src_v, dst_v, rows_v, gsem, ssem = bufs[p]
        pltpu.make_async_copy(
            hs_hbm.at[cid].at[src_v], rows_v, gsem).wait()

        def group_body(g):
            base = g * L
            srcg = src_v[pl.ds(base, L)]
            dstg = dst_v[pl.ds(base, L)]
            a = plsc.load_gather(s1_v, [srcg]) + plsc.load_gather(s2_v, [dstg])
            selu = SELU_SCALE * jnp.where(
                a > 0.0, a, SELU_ALPHA * (jnp.exp(a) - 1.0))
            alpha = 1.0 / (1.0 + jnp.exp(-selu))
            jrow = base + iota
            for col in range(DH):
                cidx = jnp.bitwise_and(col + iota, DH - 1)
                v = plsc.load_gather(rows_v, [jrow, cidx])
                plsc.store_scatter(rows_v, [jrow, cidx], v * alpha)

        plsc.parallel_loop(0, GROUPS, unroll=2)(group_body)
        pltpu.async_copy(rows_v, aggr_sh.at[dst_v], ssem, add=True)

    fetch(0, 0)
    fetch(1, 1)

    def pair_body(i2, _):
        process(0)
        process(1)

        @pl.when(i2 < N_PAIRS - 1)
        def _prefetch():
            drain_scatter(0)
            fetch(0, 2 * i2 + 2)
            drain_scatter(1)
            fetch(1, 2 * i2 + 3)

        return 0

    lax.fori_loop(0, N_PAIRS, pair_body, 0)
    drain_scatter(0)
    drain_scatter(1)
    plsc.subcore_barrier()

    pltpu.sync_copy(aggr_sh.at[pl.ds(row0, ROWS_PER_TILE)],
                    part_hbm.at[cid, pl.ds(row0, ROWS_PER_TILE)])

    @pl.when(sid == NS - 1)
    def _write_tail():
        pltpu.sync_copy(aggr_sh.at[pl.ds(NS * ROWS_PER_TILE, ROWS_LAST_EXTRA)],
                        part_hbm.at[cid, pl.ds(NS * ROWS_PER_TILE,
                                               ROWS_LAST_EXTRA)])


_sc_hop = functools.partial(
    pl.kernel,
    out_type=jax.ShapeDtypeStruct((NC, N, DH), jnp.float32),
    mesh=plsc.VectorSubcoreMesh(core_axis_name="c", subcore_axis_name="s"),
    scratch_types=[
        pltpu.VMEM((N,), jnp.float32),
        pltpu.VMEM((N,), jnp.float32),
        pltpu.VMEM((CHUNK,), jnp.int32),
        pltpu.VMEM((CHUNK,), jnp.int32),
        pltpu.VMEM((CHUNK,), jnp.int32),
        pltpu.VMEM((CHUNK,), jnp.int32),
        pltpu.VMEM((CHUNK, DH), jnp.float32),
        pltpu.VMEM((CHUNK, DH), jnp.float32),
        pltpu.VMEM_SHARED((N, DH), jnp.float32),
        pltpu.SemaphoreType.DMA,
        pltpu.SemaphoreType.DMA,
        pltpu.SemaphoreType.DMA,
        pltpu.SemaphoreType.DMA,
    ],
    compiler_params=pltpu.CompilerParams(needs_layout_passes=False,
                                         use_tc_tiling_on_sc=False),
)(_sc_hop_body)


@jax.jit
def kernel(x, edge_index, W, b):
    src = edge_index[0]
    dst = edge_index[1]
    zeros_nd = jnp.zeros((N, D), jnp.float32)
    xsplit = jnp.stack([x[:, :DH], x[:, DH:]])

    noises = [
        SIGMA * jax.random.normal(
            jax.random.fold_in(jax.random.key(1), k), (N, D), dtype=jnp.float32)
        for k in range(HOPS)
    ]

    outs = []
    p, nz = xsplit, zeros_nd
    for k in range(HOPS + 1):
        wk = W[min(k, HOPS - 1)].reshape(2, D)
        bk = b[min(k, HOPS - 1)].reshape(1, 1)
        h, hs, s1, s2 = _tc_stage(p, nz, wk, bk)
        outs.append(h)
        if k == HOPS:
            break
        p = _sc_hop(hs, s1, s2, src, dst)
        nz = noises[k]

    return jnp.stack(outs)


# speedup vs baseline: 10.3115x; 1.7003x over previous
import functools

import jax
import jax.numpy as jnp
from jax import lax
from jax.experimental import pallas as pl
from jax.experimental.pallas import tpu as pltpu
from jax.experimental.pallas import tpu_sc as plsc

N = 10000
E = 320000
D = 128
HOPS = 3
SIGMA = 0.1

NC = 2
NS = 16
L = 16
DH = D // NC

E_PER_T = E // NS
CHUNK = 400
N_CHUNKS = E_PER_T // CHUNK
N_PAIRS = N_CHUNKS // 2
GROUPS = CHUNK // L
ROWS_PER_TILE = 624
ROWS_LAST_EXTRA = N - NS * ROWS_PER_TILE

SELU_ALPHA = 1.6732632423543772
SELU_SCALE = 1.0507009873554805


def _tc_stage_body(p_ref, nz_ref, w_ref, bk_ref, h_ref, hs_ref, s1_ref,
                   s2_ref):
    agg = jnp.concatenate([p_ref[0], p_ref[1]], axis=1) + nz_ref[...]
    nrm = jnp.sqrt(jnp.sum(agg * agg, axis=1, keepdims=True))
    h = agg / jnp.maximum(nrm, 1e-12)
    h_ref[...] = h
    hs_ref[0] = h[:, :DH]
    hs_ref[1] = h[:, DH:]
    s1_ref[...] = jnp.sum(h * w_ref[0:1, :], axis=1)
    s2_ref[...] = jnp.sum(h * w_ref[1:2, :], axis=1) + bk_ref[0, 0]


def _tc_stage(p, nz, wk, bk):
    return pl.pallas_call(
        _tc_stage_body,
        out_shape=(
            jax.ShapeDtypeStruct((N, D), jnp.float32),
            jax.ShapeDtypeStruct((NC, N, DH), jnp.float32),
            jax.ShapeDtypeStruct((N,), jnp.float32),
            jax.ShapeDtypeStruct((N,), jnp.float32),
        ),
        in_specs=[
            pl.BlockSpec(memory_space=pltpu.VMEM),
            pl.BlockSpec(memory_space=pltpu.VMEM),
            pl.BlockSpec(memory_space=pltpu.VMEM),
            pl.BlockSpec(memory_space=pltpu.SMEM),
        ],
        out_specs=(
            pl.BlockSpec(memory_space=pltpu.VMEM),
            pl.BlockSpec(memory_space=pltpu.VMEM),
            pl.BlockSpec(memory_space=pltpu.VMEM),
            pl.BlockSpec(memory_space=pltpu.VMEM),
        ),
    )(p, nz, wk, bk)


def _sc_hop_body(hs_hbm, s1_hbm, s2_hbm, src_hbm, dst_hbm, part_hbm,
                 s1_v, s2_v, src0_v, dst0_v, src1_v, dst1_v,
                 rows0_v, rows1_v, alpha_v, aggr_sh,
                 gsem0, gsem1, ssem0, ssem1):
    cid = lax.axis_index("c")
    sid = lax.axis_index("s")
    bufs = ((src0_v, dst0_v, rows0_v, gsem0, ssem0),
            (src1_v, dst1_v, rows1_v, gsem1, ssem1))

    zero16 = jnp.zeros((L,), jnp.float32)

    def zbody(j, _):
        for cc in range(DH // L):
            rows0_v[j, pl.ds(cc * L, L)] = zero16
        return 0

    lax.fori_loop(0, CHUNK, zbody, 0)
    row0 = sid * ROWS_PER_TILE
    pltpu.sync_copy(rows0_v.at[pl.ds(0, CHUNK)],
                    aggr_sh.at[pl.ds(row0, CHUNK)])
    pltpu.sync_copy(rows0_v.at[pl.ds(0, ROWS_PER_TILE - CHUNK)],
                    aggr_sh.at[pl.ds(row0 + CHUNK, ROWS_PER_TILE - CHUNK)])

    @pl.when(sid == NS - 1)
    def _zero_tail():
        pltpu.sync_copy(rows0_v.at[pl.ds(0, ROWS_LAST_EXTRA)],
                        aggr_sh.at[pl.ds(NS * ROWS_PER_TILE, ROWS_LAST_EXTRA)])

    pltpu.sync_copy(s1_hbm, s1_v)
    pltpu.sync_copy(s2_hbm, s2_v)
    plsc.subcore_barrier()

    zeros_i = jnp.zeros((L,), jnp.int32)
    ebase = sid * E_PER_T

    def fetch(p, chunk_idx):
        src_v, dst_v, rows_v, gsem, _ = bufs[p]
        off = ebase + chunk_idx * CHUNK
        pltpu.sync_copy(src_hbm.at[pl.ds(off, CHUNK)], src_v)
        pltpu.sync_copy(dst_hbm.at[pl.ds(off, CHUNK)], dst_v)
        pltpu.async_copy(hs_hbm.at[cid].at[src_v],
                         rows_v, gsem)

    def drain_scatter(p):
        _, _, rows_v, _, ssem = bufs[p]
        pltpu.make_async_copy(rows_v, aggr_sh.at[pl.ds(0, CHUNK)], ssem).wait()

    def process(p):
        src_v, dst_v, rows_v, gsem, ssem = bufs[p]
        pltpu.make_async_copy(
            hs_hbm.at[cid].at[src_v], rows_v, gsem).wait()

        def alpha_body(g):
            base = g * L
            srcg = src_v[pl.ds(base, L)]
            dstg = dst_v[pl.ds(base, L)]
            a = plsc.load_gather(s1_v, [srcg]) + plsc.load_gather(s2_v, [dstg])
            selu = SELU_SCALE * jnp.where(
                a > 0.0, a, SELU_ALPHA * (jnp.exp(a) - 1.0))
            alpha_v[pl.ds(base, L)] = 1.0 / (1.0 + jnp.exp(-selu))

        plsc.parallel_loop(0, GROUPS, unroll=4)(alpha_body)

        def scale_body(j):
            av = plsc.load_gather(alpha_v, [zeros_i + j])
            for cc in range(DH // L):
                sl = pl.ds(cc * L, L)
                rows_v[j, sl] = rows_v[j, sl] * av

        plsc.parallel_loop(0, CHUNK, unroll=4)(scale_body)
        pltpu.async_copy(rows_v, aggr_sh.at[dst_v], ssem, add=True)

    fetch(0, 0)
    fetch(1, 1)

    def pair_body(i2, _):
        process(0)
        process(1)

        @pl.when(i2 < N_PAIRS - 1)
        def _prefetch():
            drain_scatter(0)
            fetch(0, 2 * i2 + 2)
            drain_scatter(1)
            fetch(1, 2 * i2 + 3)

        return 0

    lax.fori_loop(0, N_PAIRS, pair_body, 0)
    drain_scatter(0)
    drain_scatter(1)
    plsc.subcore_barrier()

    pltpu.sync_copy(aggr_sh.at[pl.ds(row0, ROWS_PER_TILE)],
                    part_hbm.at[cid, pl.ds(row0, ROWS_PER_TILE)])

    @pl.when(sid == NS - 1)
    def _write_tail():
        pltpu.sync_copy(aggr_sh.at[pl.ds(NS * ROWS_PER_TILE, ROWS_LAST_EXTRA)],
                        part_hbm.at[cid, pl.ds(NS * ROWS_PER_TILE,
                                               ROWS_LAST_EXTRA)])


_sc_hop = functools.partial(
    pl.kernel,
    out_type=jax.ShapeDtypeStruct((NC, N, DH), jnp.float32),
    mesh=plsc.VectorSubcoreMesh(core_axis_name="c", subcore_axis_name="s"),
    scratch_types=[
        pltpu.VMEM((N,), jnp.float32),
        pltpu.VMEM((N,), jnp.float32),
        pltpu.VMEM((CHUNK,), jnp.int32),
        pltpu.VMEM((CHUNK,), jnp.int32),
        pltpu.VMEM((CHUNK,), jnp.int32),
        pltpu.VMEM((CHUNK,), jnp.int32),
        pltpu.VMEM((CHUNK, DH), jnp.float32),
        pltpu.VMEM((CHUNK, DH), jnp.float32),
        pltpu.VMEM((CHUNK,), jnp.float32),
        pltpu.VMEM_SHARED((N, DH), jnp.float32),
        pltpu.SemaphoreType.DMA,
        pltpu.SemaphoreType.DMA,
        pltpu.SemaphoreType.DMA,
        pltpu.SemaphoreType.DMA,
    ],
    compiler_params=pltpu.CompilerParams(needs_layout_passes=False,
                                         use_tc_tiling_on_sc=False),
)(_sc_hop_body)


@jax.jit
def kernel(x, edge_index, W, b):
    src = edge_index[0]
    dst = edge_index[1]
    zeros_nd = jnp.zeros((N, D), jnp.float32)
    xsplit = jnp.stack([x[:, :DH], x[:, DH:]])

    noises = [
        SIGMA * jax.random.normal(
            jax.random.fold_in(jax.random.key(1), k), (N, D), dtype=jnp.float32)
        for k in range(HOPS)
    ]

    outs = []
    p, nz = xsplit, zeros_nd
    for k in range(HOPS + 1):
        wk = W[min(k, HOPS - 1)].reshape(2, D)
        bk = b[min(k, HOPS - 1)].reshape(1, 1)
        h, hs, s1, s2 = _tc_stage(p, nz, wk, bk)
        outs.append(h)
        if k == HOPS:
            break
        p = _sc_hop(hs, s1, s2, src, dst)
        nz = noises[k]

    return jnp.stack(outs)
